# Initial kernel scaffold; baseline (speedup 1.0000x reference)
import jax
import jax.numpy as jnp
from jax.experimental import pallas as pl


def kernel(s, v, edge_index_local, d_local, r_local, edge_index_global, d_global, r_global, batch, params):
    raise NotImplementedError("write your pallas kernel here")


# speedup vs baseline: 1.0000x; 1.0000x over previous
import jax
import jax.numpy as jnp
from jax.experimental import pallas as pl

N = 50000
E = 800000
SDIM = 64
VDIM = 16
RBF_DIM = 64
CUTOFF = 5.0
NUM_LAYERS = 5


def _rbf(d):
    centers = jnp.linspace(0.0, CUTOFF, RBF_DIM)
    width = CUTOFF / RBF_DIM
    return jnp.exp(-((d[:, None] - centers[None, :]) ** 2) / (2.0 * width ** 2))


def _layernorm(s, gamma, beta):
    mu = jnp.mean(s, axis=-1, keepdims=True)
    var = jnp.var(s, axis=-1, keepdims=True)
    return gamma * (s - mu) / jnp.sqrt(var + 1e-5) + beta


def _conv(s, v, edge_index, d, r, p, has_v_in, use_cutoff, use_mlp_update):
    src = edge_index[0]
    dst = edge_index[1]
    n = s.shape[0]
    h = jnp.concatenate([s[src], s[dst], _rbf(d)], axis=-1)
    h = jax.nn.silu(h @ p['W1'] + p['b1'])
    m = h @ p['W2'] + p['b2']
    ms = m[:, :SDIM]
    gv1 = m[:, SDIM:SDIM + VDIM]
    gv2 = m[:, SDIM + VDIM:]
    if use_cutoff:
        C = 0.5 * (jnp.cos(jnp.pi * jnp.clip(d, 0.0, CUTOFF) / CUTOFF) + 1.0) * (d < CUTOFF).astype(jnp.float32)
        ms = ms * C[:, None]
        gv1 = gv1 * C[:, None]
        gv2 = gv2 * C[:, None]
    r_unit = r / (jnp.linalg.norm(r, axis=-1, keepdims=True) + 1e-8)
    mv = r_unit[:, :, None] * gv2[:, None, :]
    if has_v_in:
        mv = mv + v[src] * gv1[:, None, :]
    cnt = jnp.maximum(jax.ops.segment_sum(jnp.ones((d.shape[0],), jnp.float32), dst, num_segments=n), 1.0)
    s_agg = jax.ops.segment_sum(ms, dst, num_segments=n) / cnt[:, None]
    v_agg = jax.ops.segment_sum(mv, dst, num_segments=n) / cnt[:, None, None]
    s = s + s_agg
    v = v + v_agg
    if use_mlp_update:
        s = s + jax.nn.silu(s @ p['Wu'] + p['bu'])
    return s, v


def kernel(s, v, edge_index_local, d_local, r_local, edge_index_global, d_global, r_global, batch, params):
    for i in range(NUM_LAYERS):
        use_global = (i == 0) or (i == NUM_LAYERS - 2)
        if use_global:
            ei, d, r = edge_index_global, d_global, r_global
        else:
            ei, d, r = edge_index_local, d_local, r_local
        p = params[i]
        s = _layernorm(s, p['gamma'], p['beta'])
        vn = jnp.sqrt(jnp.mean(jnp.sum(v * v, axis=1), axis=-1) + 1e-6)
        v = v / vn[:, None, None]
        s, v = _conv(s, v, ei, d, r, p, has_v_in=(i > 0), use_cutoff=(not use_global), use_mlp_update=(i < NUM_LAYERS - 1))
    return s, v


# speedup vs baseline: 13.5934x; 13.5931x over previous
import functools

import jax
import jax.numpy as jnp
from jax import lax
from jax.experimental import pallas as pl
from jax.experimental.pallas import tpu as pltpu
from jax.experimental.pallas import tpu_sc as plsc

N = 50000
E = 800000
SDIM = 64
VDIM = 16
RBF_DIM = 64
CUTOFF = 5.0
NUM_LAYERS = 5

BN = 2048
BEDGE = 3200
K = 1000

NCORES = 2
NSUB = 16
NW = NCORES * NSUB
EPW = E // NW
EPT = E // NSUB
NSTRIPE = N // NSUB


def _prep_body(s_ref, v_ref, g_ref, b_ref, w1a_ref, w1b_ref,
               sln_ref, t1_ref, t2_ref, vh_ref):
    s = s_ref[...]
    mu = jnp.mean(s, axis=-1, keepdims=True)
    var = jnp.mean((s - mu) ** 2, axis=-1, keepdims=True)
    sln = g_ref[...] * (s - mu) / jnp.sqrt(var + 1e-5) + b_ref[...]
    sln_ref[...] = sln
    a = jnp.dot(sln, w1a_ref[...], preferred_element_type=jnp.float32)
    b = jnp.dot(sln, w1b_ref[...], preferred_element_type=jnp.float32)
    v = v_ref[...]
    v2 = v * v
    vsum = v2[:, 0:16] + v2[:, 16:32] + v2[:, 32:48]
    vn = jnp.sqrt(jnp.mean(vsum, axis=-1, keepdims=True) + 1e-6)
    vh = v / vn
    vh_ref[...] = vh
    pad16 = jnp.zeros((a.shape[0], 16), jnp.float32)
    pad64 = jnp.zeros((a.shape[0], 64), jnp.float32)
    t1_ref[...] = jnp.concatenate([a, vh, pad16], axis=1)
    t2_ref[...] = jnp.concatenate([b, pad64], axis=1)


def _tc_prep(s, v2, gamma, beta, w1a, w1b):
    grid = ((N + BN - 1) // BN,)
    blk = lambda w: pl.BlockSpec((BN, w), lambda i: (i, 0))
    cst = lambda a, b: pl.BlockSpec((a, b), lambda i: (0, 0))
    return pl.pallas_call(
        _prep_body,
        grid=grid,
        in_specs=[blk(64), blk(48), cst(1, 64), cst(1, 64), cst(64, 64), cst(64, 64)],
        out_specs=[blk(64), blk(128), blk(128), blk(48)],
        out_shape=[
            jax.ShapeDtypeStruct((N, SDIM), jnp.float32),
            jax.ShapeDtypeStruct((N, 128), jnp.float32),
            jax.ShapeDtypeStruct((N, 128), jnp.float32),
            jax.ShapeDtypeStruct((N, 48), jnp.float32),
        ],
    )(s, v2, gamma, beta, w1a, w1b)


def _edge_body_common(ga, gb, vs, d, r, w1c, b1, w2, b2, use_cutoff):
    centers = lax.broadcasted_iota(jnp.int32, (1, RBF_DIM), 1).astype(jnp.float32) * (CUTOFF / (RBF_DIM - 1))
    width = CUTOFF / RBF_DIM
    rbf = jnp.exp(-((d - centers) ** 2) * (1.0 / (2.0 * width * width)))
    h = ga + gb + jnp.dot(rbf, w1c, preferred_element_type=jnp.float32) + b1
    h = h * jax.nn.sigmoid(h)
    m = jnp.dot(h, w2, preferred_element_type=jnp.float32) + b2
    ms = m[:, :SDIM]
    gv1 = m[:, SDIM:SDIM + VDIM]
    gv2 = m[:, SDIM + VDIM:]
    if use_cutoff:
        c = 0.5 * (jnp.cos(jnp.pi * jnp.clip(d, 0.0, CUTOFF) / CUTOFF) + 1.0)
        c = c * (d < CUTOFF).astype(jnp.float32)
        ms = ms * c
        gv1 = gv1 * c
        gv2 = gv2 * c
    rn = jnp.sqrt(jnp.sum(r * r, axis=-1, keepdims=True))
    ru = r / (rn + 1e-8)
    mv = jnp.concatenate([ru[:, 0:1] * gv2, ru[:, 1:2] * gv2, ru[:, 2:3] * gv2], axis=1)
    if vs is not None:
        mv = mv + vs * jnp.concatenate([gv1, gv1, gv1], axis=1)
    return ms, mv


def _edge_body(g1_ref, g2_ref, d_ref, r_ref, w1c_ref, b1_ref, w2_ref, b2_ref,
               mst0_ref, mst1_ref, mvt0_ref, mvt1_ref, *, use_cutoff, has_v):
    g1 = g1_ref[...]
    vs = g1[:, 64:112] if has_v else None
    ms, mv = _edge_body_common(g1[:, 0:64], g2_ref[:, 0:64], vs, d_ref[...],
                               r_ref[...], w1c_ref[...], b1_ref[...], w2_ref[...],
                               b2_ref[...], use_cutoff)
    mst = jnp.transpose(ms)
    mvt = jnp.transpose(mv)
    mst0_ref[...] = mst[:32, :]
    mst1_ref[...] = mst[32:, :]
    mvt0_ref[...] = mvt[:24, :]
    mvt1_ref[...] = mvt[24:, :]


def _tc_edge(g1, g2, d2, r, w1c, b1, w2, b2, use_cutoff, has_v):
    grid = (E // BEDGE,)
    blk = lambda w: pl.BlockSpec((BEDGE, w), lambda i: (i, 0))
    tblk = lambda w: pl.BlockSpec((w, BEDGE), lambda i: (0, i))
    cst = lambda a, b: pl.BlockSpec((a, b), lambda i: (0, 0))
    out_shape = [
        jax.ShapeDtypeStruct((32, EP), jnp.float32),
        jax.ShapeDtypeStruct((32, EP), jnp.float32),
        jax.ShapeDtypeStruct((24, EP), jnp.float32),
        jax.ShapeDtypeStruct((24, EP), jnp.float32),
    ]
    out_specs = [tblk(32), tblk(32), tblk(24), tblk(24)]
    wspecs = [cst(64, 64), cst(1, 64), cst(64, 96), cst(1, 96)]
    body = functools.partial(_edge_body, use_cutoff=use_cutoff, has_v=has_v)
    in_specs = [blk(128), blk(128), blk(1), blk(3)] + wspecs
    return pl.pallas_call(
        body, grid=grid, in_specs=in_specs, out_specs=out_specs, out_shape=out_shape,
    )(g1, g2, d2, r, w1c, b1, w2, b2)


def _update_body(sln_ref, vh_ref, sa0a_ref, sa0b_ref, sa1a_ref, sa1b_ref,
                 va0a_ref, va0b_ref, va1a_ref, va1b_ref,
                 ca_ref, cb_ref, wu_ref, bu_ref, s_ref, v_ref, *, use_mlp):
    cnt = jnp.maximum(ca_ref[...] + cb_ref[...], 1.0)
    sagg = jnp.concatenate([jnp.transpose(sa0a_ref[...] + sa0b_ref[...]),
                            jnp.transpose(sa1a_ref[...] + sa1b_ref[...])], axis=1)
    vagg = jnp.concatenate([jnp.transpose(va0a_ref[...] + va0b_ref[...]),
                            jnp.transpose(va1a_ref[...] + va1b_ref[...])], axis=1)
    s = sln_ref[...] + sagg / cnt
    if use_mlp:
        u = jnp.dot(s, wu_ref[...], preferred_element_type=jnp.float32) + bu_ref[...]
        s = s + u * jax.nn.sigmoid(u)
    s_ref[...] = s
    v_ref[...] = vh_ref[...] + vagg / cnt


def _tc_update(sln, vh, aggs, cnta, cntb, wu, bu, use_mlp):
    grid = ((N + BN - 1) // BN,)
    blk = lambda w: pl.BlockSpec((BN, w), lambda i: (i, 0))
    tblk = lambda w: pl.BlockSpec((w, BN), lambda i: (0, i))
    cst = lambda a, b: pl.BlockSpec((a, b), lambda i: (0, 0))
    return pl.pallas_call(
        functools.partial(_update_body, use_mlp=use_mlp),
        grid=grid,
        in_specs=[blk(64), blk(48), tblk(32), tblk(32), tblk(32), tblk(32),
                  tblk(24), tblk(24), tblk(24), tblk(24),
                  blk(1), blk(1), cst(64, 64), cst(1, 64)],
        out_specs=[blk(64), blk(48)],
        out_shape=[
            jax.ShapeDtypeStruct((N, SDIM), jnp.float32),
            jax.ShapeDtypeStruct((N, 48), jnp.float32),
        ],
    )(sln, vh, *aggs, cnta, cntb, wu, bu)


@functools.cache
def _mesh():
    return plsc.VectorSubcoreMesh(core_axis_name="c", subcore_axis_name="s",
                                  num_cores=NCORES, num_subcores=NSUB)


def _worker_id():
    return lax.axis_index("s") * NCORES + lax.axis_index("c")


EP = 819200
NROWS = EP // 128
RPW = NROWS // NW
RPT = NROWS // NSUB
RB = 40
NPAD = 128


def _gather_body(t1_hbm, t2_hbm, srcp_hbm, dstp_hbm,
                 g1_hbm, g2_hbm, idx1, idx2, buf1, buf2, sem1, sem2):
    w = _worker_id()
    row0 = w * RPW

    def outer(b, _):
        rb = row0 + b * RB
        pltpu.sync_copy(srcp_hbm.at[pl.ds(rb, RB)], idx1)
        pltpu.sync_copy(dstp_hbm.at[pl.ds(rb, RB)], idx2)

        def inner(j, _):
            erow = pl.multiple_of((rb + j) * 128, 128)
            cp1 = pltpu.async_copy(t1_hbm.at[idx1.at[j]], buf1, sem1)
            cp2 = pltpu.async_copy(t2_hbm.at[idx2.at[j]], buf2, sem2)
            cp1.wait()
            pltpu.sync_copy(buf1, g1_hbm.at[pl.ds(erow, 128)])
            cp2.wait()
            pltpu.sync_copy(buf2, g2_hbm.at[pl.ds(erow, 128)])
            return _

        lax.fori_loop(0, RB, inner, None)
        return _

    lax.fori_loop(0, RPW // RB, outer, None)


def _sc_gather(t1, t2, srcp, dstp):
    fn = pl.kernel(
        _gather_body,
        out_type=[
            jax.ShapeDtypeStruct((EP, 128), jnp.float32),
            jax.ShapeDtypeStruct((EP, 128), jnp.float32),
        ],
        mesh=_mesh(),
        scratch_types=[
            pltpu.VMEM((RB, 128), jnp.int32),
            pltpu.VMEM((RB, 128), jnp.int32),
            pltpu.VMEM((128, 128), jnp.float32),
            pltpu.VMEM((128, 128), jnp.float32),
            pltpu.SemaphoreType.DMA,
            pltpu.SemaphoreType.DMA,
        ],
    )
    return fn(t1, t2, srcp, dstp)


NSTR2 = 3200
NACC = NSTR2 * NSUB
NPAD = NACC - N


def _scatcol_body(dstp_hbm, updt_hbm, zeros_hbm, outa_hbm, outb_hbm,
                  acc, idx, ubuf, zbuf, *, ncols):
    c = lax.axis_index("c")
    t = lax.axis_index("s")
    pltpu.sync_copy(zeros_hbm, zbuf)
    zlen = ncols * NACC // NSUB

    def zstep(j, _):
        pltpu.sync_copy(zbuf, acc.at[pl.ds(pl.multiple_of(t * zlen + j * NSTR2, 8), NSTR2)])
        return _

    lax.fori_loop(0, zlen // NSTR2, zstep, None)
    plsc.subcore_barrier()
    row0 = _worker_id() * RPW

    def phase(cp, _):
        accc = acc.at[pl.ds(pl.multiple_of(cp * NACC, 8), NACC)]

        def block(b, _):
            rb = row0 + b * RB
            pltpu.sync_copy(dstp_hbm.at[pl.ds(rb, RB)], idx)
            pltpu.sync_copy(updt_hbm.at[cp, pl.ds(rb, RB)], ubuf)

            def row(j, _):
                pltpu.sync_copy(ubuf.at[j], accc.at[idx.at[j]], add=True)
                return _

            lax.fori_loop(0, RB, row, None)
            return _

        lax.fori_loop(0, RPW // RB, block, None)
        return _

    lax.fori_loop(0, ncols, phase, None)
    plsc.subcore_barrier()

    def wstep(j, _):
        q0 = pl.multiple_of(t * zlen + j * NSTR2, 8)
        pltpu.sync_copy(acc.at[pl.ds(q0, NSTR2)], zbuf)

        @pl.when(c == 0)
        def _():
            pltpu.sync_copy(zbuf, outa_hbm.at[pl.ds(q0, NSTR2)])

        @pl.when(c == 1)
        def _():
            pltpu.sync_copy(zbuf, outb_hbm.at[pl.ds(q0, NSTR2)])

        return _

    lax.fori_loop(0, zlen // NSTR2, wstep, None)


def _sc_scatter(dstp, updt, ncols):
    zeros = jnp.zeros((NSTR2,), jnp.float32)
    fn = pl.kernel(
        functools.partial(_scatcol_body, ncols=ncols),
        out_type=[
            jax.ShapeDtypeStruct((ncols * NACC,), jnp.float32),
            jax.ShapeDtypeStruct((ncols * NACC,), jnp.float32),
        ],
        mesh=_mesh(),
        scratch_types=[
            pltpu.VMEM_SHARED((ncols * NACC,), jnp.float32),
            pltpu.VMEM((RB, 128), jnp.int32),
            pltpu.VMEM((RB, 128), jnp.float32),
            pltpu.VMEM((NSTR2,), jnp.float32),
        ],
    )
    oa, ob = fn(dstp, updt, zeros)
    return oa.reshape(ncols, NACC), ob.reshape(ncols, NACC)


def _count_body(dstp_hbm, zeros_hbm, ones_hbm, outa_hbm, outb_hbm,
                acc, idx, ones_v, cbuf):
    c = lax.axis_index("c")
    t = lax.axis_index("s")
    z0 = pl.multiple_of(t * NSTR2, 8)
    pltpu.sync_copy(zeros_hbm, cbuf)
    pltpu.sync_copy(cbuf, acc.at[pl.ds(z0, NSTR2)])
    pltpu.sync_copy(ones_hbm, ones_v)
    plsc.subcore_barrier()
    row0 = _worker_id() * RPW

    def outer(b, _):
        rb = row0 + b * RB
        pltpu.sync_copy(dstp_hbm.at[pl.ds(rb, RB)], idx)

        def inner(j, _):
            pltpu.sync_copy(ones_v, acc.at[idx.at[j]], add=True)
            return _

        lax.fori_loop(0, RB, inner, None)
        return _

    lax.fori_loop(0, RPW // RB, outer, None)
    plsc.subcore_barrier()
    pltpu.sync_copy(acc.at[pl.ds(z0, NSTR2)], cbuf)

    @pl.when(c == 0)
    def _():
        pltpu.sync_copy(cbuf, outa_hbm.at[pl.ds(z0, NSTR2)])

    @pl.when(c == 1)
    def _():
        pltpu.sync_copy(cbuf, outb_hbm.at[pl.ds(z0, NSTR2)])


def _sc_count(dstp):
    zeros = jnp.zeros((NSTR2,), jnp.float32)
    ones = jnp.ones((128,), jnp.float32)
    fn = pl.kernel(
        _count_body,
        out_type=[
            jax.ShapeDtypeStruct((NACC,), jnp.float32),
            jax.ShapeDtypeStruct((NACC,), jnp.float32),
        ],
        mesh=_mesh(),
        scratch_types=[
            pltpu.VMEM_SHARED((NACC,), jnp.float32),
            pltpu.VMEM((RB, 128), jnp.int32),
            pltpu.VMEM((128,), jnp.float32),
            pltpu.VMEM((NSTR2,), jnp.float32),
        ],
    )
    return fn(dstp, zeros, ones)


def kernel(s, v, edge_index_local, d_local, r_local, edge_index_global,
           d_global, r_global, batch, params):
    v2 = v.reshape(N, 48)
    pad_g = (jnp.arange(EP - E, dtype=jnp.int32) % 128)
    pad_s = N + (jnp.arange(EP - E, dtype=jnp.int32) % NPAD)

    def padg(ix):
        return jnp.concatenate([ix, pad_g]).reshape(NROWS, 128)

    def pads(ix):
        return jnp.concatenate([ix, pad_s]).reshape(NROWS, 128)

    src_l = padg(edge_index_local[0])
    dst_l = padg(edge_index_local[1])
    src_g = padg(edge_index_global[0])
    dst_g = padg(edge_index_global[1])
    dsts_l = pads(edge_index_local[1])
    dsts_g = pads(edge_index_global[1])
    d2_l = d_local[:, None]
    d2_g = d_global[:, None]

    ca_l, cb_l = _sc_count(dsts_l)
    ca_g, cb_g = _sc_count(dsts_g)
    cnt_l = (ca_l[:, None], cb_l[:, None])
    cnt_g = (ca_g[:, None], cb_g[:, None])

    for i in range(NUM_LAYERS):
        use_global = (i == 0) or (i == NUM_LAYERS - 2)
        if use_global:
            src, dst, dsts, d2, r, cnt = src_g, dst_g, dsts_g, d2_g, r_global, cnt_g
        else:
            src, dst, dsts, d2, r, cnt = src_l, dst_l, dsts_l, d2_l, r_local, cnt_l
        p = params[i]
        has_v = i > 0
        use_cutoff = not use_global
        use_mlp = i < NUM_LAYERS - 1

        w1 = p['W1']
        w1a, w1b, w1c = w1[:64], w1[64:128], w1[128:]
        gamma = p['gamma'][None, :]
        beta = p['beta'][None, :]
        b1 = p['b1'][None, :]
        b2 = p['b2'][None, :]
        bu = p['bu'][None, :]

        sln, t1, t2, vh = _tc_prep(s, v2, gamma, beta, w1a, w1b)
        g1, g2 = _sc_gather(t1, t2, src, dst)
        mst0, mst1, mvt0, mvt1 = _tc_edge(g1, g2, d2, r, w1c, b1, p['W2'], b2,
                                          use_cutoff, has_v)
        sa0a, sa0b = _sc_scatter(dsts, mst0.reshape(32, NROWS, 128), 32)
        sa1a, sa1b = _sc_scatter(dsts, mst1.reshape(32, NROWS, 128), 32)
        va0a, va0b = _sc_scatter(dsts, mvt0.reshape(24, NROWS, 128), 24)
        va1a, va1b = _sc_scatter(dsts, mvt1.reshape(24, NROWS, 128), 24)
        aggs = (sa0a, sa0b, sa1a, sa1b, va0a, va0b, va1a, va1b)
        s, v2 = _tc_update(sln, vh, aggs, cnt[0], cnt[1], p['Wu'], bu, use_mlp)

    return s, v2.reshape(N, 3, VDIM)
